# Initial kernel scaffold; baseline (speedup 1.0000x reference)
import jax
import jax.numpy as jnp
from jax.experimental import pallas as pl


def kernel(t, emb):
    raise NotImplementedError("write your pallas kernel here")


# speedup vs baseline: 3.2348x; 3.2348x over previous
import functools

import jax
import jax.numpy as jnp
from jax import lax
from jax.experimental import pallas as pl
from jax.experimental.pallas import tpu as pltpu
from jax.experimental.pallas import tpu_sc as plsc

_MAX_TIME = 1.0
_TIME_NUM = 100000

_NC, _NS = 2, 16
_NW = _NC * _NS
_G = 128
_NBUF = 2


def _sc_body(n_chunks, t_hbm, emb_hbm, out_hbm,
             t_v, idx_v, rows, gsems, ssems):
    b_per_w = n_chunks * _G
    wid = lax.axis_index("s") * _NC + lax.axis_index("c")
    base = wid * b_per_w

    pltpu.sync_copy(t_hbm.at[pl.ds(base, b_per_w)], t_v)

    scale = float(_TIME_NUM - 1) / float(_MAX_TIME)

    def quant(j, _):
        for i in range(_G // 16):
            v = t_v[pl.ds(j * _G + i * 16, 16)]
            idx_v[j, pl.ds(i * 16, 16)] = (v * scale).astype(jnp.int32)
        return 0

    lax.fori_loop(0, n_chunks, quant, 0)

    n_outer = n_chunks // _NBUF

    def fire_gather(chunk, b):
        return pltpu.async_copy(emb_hbm.at[idx_v.at[chunk]], rows[b], gsems[b])

    for b in range(_NBUF):
        fire_gather(b, b)

    def outer(p, _):
        g0 = p * _NBUF
        for b in range(_NBUF):
            pltpu.make_async_copy(emb_hbm.at[idx_v.at[g0 + b]], rows[b],
                                  gsems[b]).wait()
            pltpu.async_copy(rows[b],
                             out_hbm.at[pl.ds(base + (g0 + b) * _G, _G)],
                             ssems[b])
        for b in range(_NBUF):
            @pl.when(p + 1 < n_outer)
            def _():
                pltpu.make_async_copy(rows[b],
                                      out_hbm.at[pl.ds(base + (g0 + b) * _G, _G)],
                                      ssems[b]).wait()
                fire_gather(g0 + _NBUF + b, b)
        return 0

    lax.fori_loop(0, n_outer, outer, 0)

    last0 = (n_outer - 1) * _NBUF
    for b in range(_NBUF):
        pltpu.make_async_copy(rows[b],
                              out_hbm.at[pl.ds(base + (last0 + b) * _G, _G)],
                              ssems[b]).wait()


def _build(B, V, D):
    assert B % (_NW * _G) == 0
    b_per_w = B // _NW
    n_chunks = b_per_w // _G
    mesh = plsc.VectorSubcoreMesh(core_axis_name="c", subcore_axis_name="s",
                                  num_cores=_NC, num_subcores=_NS)
    scratch = [
        pltpu.VMEM((n_chunks * _G,), jnp.float32),
        pltpu.VMEM((n_chunks, _G), jnp.int32),
        tuple(pltpu.VMEM((_G, D), jnp.float32) for _ in range(_NBUF)),
        tuple(pltpu.SemaphoreType.DMA for _ in range(_NBUF)),
        tuple(pltpu.SemaphoreType.DMA for _ in range(_NBUF)),
    ]
    return pl.kernel(
        functools.partial(_sc_body, n_chunks),
        out_type=jax.ShapeDtypeStruct((B, D), jnp.float32),
        mesh=mesh,
        scratch_types=scratch,
    )


@jax.jit
def kernel(t, emb):
    B = t.size
    V, D = emb.shape
    out = _build(B, V, D)(t.reshape(B), emb)
    return out.reshape(*t.shape, D)
